# baseline (speedup 1.0000x reference)
import functools

import jax
import jax.numpy as jnp
from jax import lax
from jax.experimental import pallas as pl
from jax.experimental.pallas import tpu as pltpu
from jax.experimental.pallas import tpu_sc as plsc
from jax._src.pallas import mpmd

N_CLASSES = 1000
ZBLK = 1000


@functools.partial(jax.jit, static_argnums=(0, 1))
def _zero2d_tc(rows, cols):
    def body(out_ref):
        out_ref[...] = jnp.zeros((ZBLK, cols), jnp.float32)

    return pl.pallas_call(
        body,
        grid=(rows // ZBLK,),
        out_specs=pl.BlockSpec((ZBLK, cols), lambda g: (g, 0)),
        out_shape=jax.ShapeDtypeStruct((rows, cols), jnp.float32),
    )()


@functools.partial(jax.jit, static_argnums=(2, 3, 4))
def _ones_sc(z2d, xi, b0, b1, n):
    info = plsc.get_sparse_core_info()
    nc, ns, lanes = info.num_cores, info.num_subcores, info.num_lanes
    nw = nc * ns
    i_per_w = b0 // nw
    assert b0 == nw * i_per_w and i_per_w % lanes == 0
    nchunk = i_per_w // lanes
    vals_per_w = i_per_w * b1
    cvals = lanes * b1

    mesh = plsc.VectorSubcoreMesh(core_axis_name="c", subcore_axis_name="s")

    def body(out_hbm, x_hbm, xv, slots, idx_slots, dummy, sem_d):
        wid = lax.axis_index("s") * nc + lax.axis_index("c")
        vbase = wid * vals_per_w
        i0 = wid * i_per_w
        pltpu.sync_copy(x_hbm.at[pl.ds(vbase, vals_per_w)], xv)

        iv = lax.iota(jnp.int32, lanes)
        one16 = jnp.full((lanes,), 1.0, jnp.float32)
        zero16 = jnp.zeros((lanes,), jnp.float32)

        def drain(k, _):
            pltpu.make_async_copy(
                x_hbm.at[pl.ds(0, i_per_w * i_per_w)], dummy, sem_d
            ).wait()
            return 0

        def jbody(j, _):
            jp = lax.rem(j, 4)

            @pl.when(j >= 4)
            def _():
                lax.fori_loop(0, 1, drain, 0)

            vv = [
                plsc.load_gather(xv, [(u * lanes + iv) * b1 + j])
                for u in range(nchunk)
            ]
            for u in range(nchunk):
                idx_slots[jp, pl.ds(u * lanes, lanes)] = j * n + vv[u]

            @plsc.parallel_loop(0, i_per_w, unroll=4)
            def iibody(ii):
                vii = plsc.load_gather(
                    xv, [jnp.full((lanes,), ii * b1 + j, jnp.int32)]
                )
                for u in range(nchunk):
                    slots[jp, ii, pl.ds(u * lanes, lanes)] = jnp.where(
                        vv[u] == vii, one16, zero16
                    )
            pltpu.make_async_copy(
                slots.at[jp],
                out_hbm.at[idx_slots.at[jp], pl.ds(i0, i_per_w)],
                sem_d,
            ).start()
            return 0

        lax.fori_loop(0, b1, jbody, 0)
        lax.fori_loop(0, 4, drain, 0)

    k2 = mpmd.mpmd_map(
        [(mesh, body)],
        out_types=(),
        compiler_params=pltpu.CompilerParams(
            needs_layout_passes=False, use_tc_tiling_on_sc=True
        ),
        scratch_types=[
            pltpu.VMEM((vals_per_w,), jnp.int32),
            pltpu.VMEM((4, i_per_w, i_per_w), jnp.float32),
            pltpu.VMEM((4, i_per_w), jnp.int32),
            pltpu.VMEM((i_per_w * i_per_w,), jnp.int32),
            pltpu.SemaphoreType.DMA,
        ],
    )
    buf = jax.new_ref(z2d)
    k2(buf, xi)
    return jax.freeze(buf)


def kernel(x):
    b0, b1 = x.shape
    n = N_CLASSES
    xi = x.reshape(b0 * b1).astype(jnp.int32)
    z2d = _zero2d_tc(b1 * n, b0)
    out2d = _ones_sc(z2d, xi, b0, b1, n)
    return jnp.transpose(out2d.reshape(b1, n, b0), (2, 0, 1))


# speedup vs baseline: 1.0392x; 1.0392x over previous
import functools

import jax
import jax.numpy as jnp
from jax import lax
from jax.experimental import pallas as pl
from jax.experimental.pallas import tpu as pltpu
from jax.experimental.pallas import tpu_sc as plsc
from jax._src.pallas import mpmd

N_CLASSES = 1000
ZBLK = 200


@functools.partial(jax.jit, static_argnums=(0, 1))
def _zero2d_tc(rows, cols):
    def body(out_ref):
        out_ref[...] = jnp.zeros((ZBLK, cols), jnp.float32)

    return pl.pallas_call(
        body,
        grid=(rows // ZBLK,),
        out_specs=pl.BlockSpec((ZBLK, cols), lambda g: (g, 0)),
        out_shape=jax.ShapeDtypeStruct((rows, cols), jnp.float32),
    )()


@functools.partial(jax.jit, static_argnums=(2, 3, 4))
def _ones_sc(z2d, xi, b0, b1, n):
    info = plsc.get_sparse_core_info()
    nc, ns, lanes = info.num_cores, info.num_subcores, info.num_lanes
    nw = nc * ns
    i_per_w = b0 // nw
    assert b0 == nw * i_per_w and i_per_w % lanes == 0
    nchunk = i_per_w // lanes
    vals_per_w = i_per_w * b1
    cvals = lanes * b1

    mesh = plsc.VectorSubcoreMesh(core_axis_name="c", subcore_axis_name="s")

    def body(out_hbm, x_hbm, xv, slots, idx_slots, dummy, sem_d):
        wid = lax.axis_index("s") * nc + lax.axis_index("c")
        vbase = wid * vals_per_w
        i0 = wid * i_per_w
        pltpu.sync_copy(x_hbm.at[pl.ds(vbase, vals_per_w)], xv)

        iv = lax.iota(jnp.int32, lanes)
        one16 = jnp.full((lanes,), 1.0, jnp.float32)
        zero16 = jnp.zeros((lanes,), jnp.float32)

        def drain(k, _):
            pltpu.make_async_copy(
                x_hbm.at[pl.ds(0, i_per_w * i_per_w)], dummy, sem_d
            ).wait()
            return 0

        def jbody(j, _):
            jp = lax.rem(j, 4)

            @pl.when(j >= 4)
            def _():
                lax.fori_loop(0, 1, drain, 0)

            vv = [
                plsc.load_gather(xv, [(u * lanes + iv) * b1 + j])
                for u in range(nchunk)
            ]
            for u in range(nchunk):
                idx_slots[jp, pl.ds(u * lanes, lanes)] = j * n + vv[u]

            @plsc.parallel_loop(0, i_per_w, unroll=4)
            def iibody(ii):
                vii = plsc.load_gather(
                    xv, [jnp.full((lanes,), ii * b1 + j, jnp.int32)]
                )
                for u in range(nchunk):
                    slots[jp, ii, pl.ds(u * lanes, lanes)] = jnp.where(
                        vv[u] == vii, one16, zero16
                    )
            pltpu.make_async_copy(
                slots.at[jp],
                out_hbm.at[idx_slots.at[jp], pl.ds(i0, i_per_w)],
                sem_d,
            ).start()
            return 0

        lax.fori_loop(0, b1, jbody, 0)
        lax.fori_loop(0, 4, drain, 0)

    k2 = mpmd.mpmd_map(
        [(mesh, body)],
        out_types=(),
        compiler_params=pltpu.CompilerParams(
            needs_layout_passes=False, use_tc_tiling_on_sc=True
        ),
        scratch_types=[
            pltpu.VMEM((vals_per_w,), jnp.int32),
            pltpu.VMEM((4, i_per_w, i_per_w), jnp.float32),
            pltpu.VMEM((4, i_per_w), jnp.int32),
            pltpu.VMEM((i_per_w * i_per_w,), jnp.int32),
            pltpu.SemaphoreType.DMA,
        ],
    )
    buf = jax.new_ref(z2d)
    k2(buf, xi)
    return jax.freeze(buf)


def kernel(x):
    b0, b1 = x.shape
    n = N_CLASSES
    xi = x.reshape(b0 * b1).astype(jnp.int32)
    z2d = _zero2d_tc(b1 * n, b0)
    out2d = _ones_sc(z2d, xi, b0, b1, n)
    return jnp.transpose(out2d.reshape(b1, n, b0), (2, 0, 1))
